# baseline (speedup 1.0000x reference)
import functools

import jax
import jax.numpy as jnp
from jax import lax
from jax.experimental import pallas as pl
from jax.experimental.pallas import tpu as pltpu
from jax.experimental.pallas import tpu_sc as plsc

F32 = jnp.float32
NC, NS = 2, 16
NW = NC * NS
LANES = 16


def _sc_mesh():
    return plsc.VectorSubcoreMesh(
        core_axis_name="c", subcore_axis_name="s",
        num_cores=NC, num_subcores=NS)


def _sc_counts(d1, d2, n1, n2):
    NH = n1 + n2
    E1, E2 = d1.shape[0] // NW, d2.shape[0] // NW
    assert E1 % LANES == 0 and E2 % LANES == 0 and NH % LANES == 0

    @functools.partial(
        pl.kernel,
        out_type=jax.ShapeDtypeStruct((NW, NH), F32),
        mesh=_sc_mesh(),
        scratch_types=[pltpu.VMEM((E1,), jnp.int32),
                       pltpu.VMEM((E2,), jnp.int32),
                       pltpu.VMEM((NH,), F32),
                       pltpu.SemaphoreType.DMA],
        compiler_params=pltpu.CompilerParams(needs_layout_passes=False),
    )
    def k(d1_hbm, d2_hbm, out_hbm, idx1, idx2, hist, sem):
        wid = lax.axis_index("s") * NC + lax.axis_index("c")
        pltpu.async_copy(d1_hbm.at[pl.ds(wid * E1, E1)], idx1, sem)
        pltpu.async_copy(d2_hbm.at[pl.ds(wid * E2, E2)], idx2, sem)

        def zero(i, _):
            hist[pl.ds(i * LANES, LANES)] = jnp.zeros((LANES,), F32)
            return _
        lax.fori_loop(0, NH // LANES, zero, None)

        pltpu.make_async_copy(d1_hbm.at[pl.ds(wid * E1, E1)], idx1,
                              sem).wait()
        pltpu.make_async_copy(d2_hbm.at[pl.ds(wid * E2, E2)], idx2,
                              sem).wait()
        ones = jnp.ones((LANES,), F32)
        for (idxb, eper, off) in ((idx1, E1, 0), (idx2, E2, n1)):
            def inner(j, _, idxb=idxb, off=off):
                iv = idxb[pl.ds(j * LANES, LANES)]
                if off:
                    iv = iv + off
                plsc.addupdate_scatter(hist, [iv], ones)
                return _
            lax.fori_loop(0, eper // LANES, inner, None)

        pltpu.sync_copy(hist, out_hbm.at[wid])

    return k(d1, d2)


def _prep(hists, x_pad):
    n = x_pad.shape[0]
    BN = 256
    assert n % BN == 0

    def body(h_ref, x_ref, ys_ref, yr_ref, dv_ref):
        cnt = jnp.sum(h_ref[...], axis=0)
        dv = jnp.where(cnt > 0, lax.rsqrt(cnt), 0.0)
        x = x_ref[...]
        is112 = lax.broadcasted_iota(jnp.int32, x.shape, 1) == 112
        yr_ref[...] = jnp.where(is112, 1.0, x)
        ys_ref[...] = jnp.where(is112, dv[:, None], x * dv[:, None])
        dv_ref[...] = jnp.broadcast_to(dv[None, :], (8, BN))

    return pl.pallas_call(
        body,
        grid=(n // BN,),
        in_specs=[pl.BlockSpec((NW, BN), lambda b: (0, b)),
                  pl.BlockSpec((BN, 128), lambda b: (b, 0))],
        out_specs=[pl.BlockSpec((BN, 128), lambda b: (b, 0)),
                   pl.BlockSpec((BN, 128), lambda b: (b, 0)),
                   pl.BlockSpec((8, BN), lambda b: (0, b))],
        out_shape=[jax.ShapeDtypeStruct((n, 128), F32),
                   jax.ShapeDtypeStruct((n, 128), F32),
                   jax.ShapeDtypeStruct((8, n), F32)],
    )(hists, x_pad)


def _sc_spmm_one(table, src, dst, ndst):
    E = src.shape[0]
    eper = E // NW
    C = 128
    K = eper // C
    TL = eper - K * C
    NB4 = 2
    P = (K + NB4 - 1) // NB4
    assert E % NW == 0 and TL % 8 == 0 and ndst % (NS * 32) == 0

    scratch = []
    for _ in range(NB4):
        scratch += [pltpu.VMEM((C,), jnp.int32),
                    pltpu.VMEM((C,), jnp.int32),
                    pltpu.VMEM((C, 128), F32)]
    scratch += [pltpu.VMEM((max(TL, 8),), jnp.int32),
                pltpu.VMEM((max(TL, 8),), jnp.int32),
                pltpu.VMEM((32, 128), F32),
                pltpu.VMEM_SHARED((ndst, 128), F32)]
    scratch += [pltpu.SemaphoreType.DMA] * (2 * NB4 + 1)

    @functools.partial(
        pl.kernel,
        out_type=jax.ShapeDtypeStruct((NC, ndst, 128), F32),
        mesh=_sc_mesh(),
        scratch_types=scratch,
    )
    def k(tab, s_h, d_h, out, *sc):
        sidx = [sc[3 * q + 0] for q in range(NB4)]
        didx = [sc[3 * q + 1] for q in range(NB4)]
        rows = [sc[3 * q + 2] for q in range(NB4)]
        sT, dT, zbuf, accsh = sc[3 * NB4:3 * NB4 + 4]
        gsem = list(sc[3 * NB4 + 4:3 * NB4 + 4 + NB4])
        ssem = list(sc[3 * NB4 + 4 + NB4:3 * NB4 + 4 + 2 * NB4])
        osem = sc[3 * NB4 + 4 + 2 * NB4]

        cid = lax.axis_index("c")
        sid = lax.axis_index("s")
        wid = sid * NC + cid
        e0 = wid * eper

        def stage(j, q):
            pltpu.sync_copy(s_h.at[pl.ds(e0 + j * C, C)], sidx[q])
            pltpu.sync_copy(d_h.at[pl.ds(e0 + j * C, C)], didx[q])
            pltpu.async_copy(tab.at[sidx[q]], rows[q], gsem[q])

        for q in range(min(NB4, K)):
            stage(q, q)

        def zb(i, _):
            zbuf[i // 8, pl.ds((i % 8) * LANES, LANES)] = (
                jnp.zeros((LANES,), F32))
            return _
        lax.fori_loop(0, 256, zb, None)

        rpt = ndst // NS
        r0 = sid * rpt

        def zl(i, _):
            pltpu.sync_copy(zbuf, accsh.at[pl.ds(r0 + i * 32, 32)])
            return _
        lax.fori_loop(0, rpt // 32, zl, None)
        plsc.subcore_barrier()

        def wait_g(q):
            pltpu.make_async_copy(tab.at[sidx[q]], rows[q], gsem[q]).wait()

        def wait_s(q):
            pltpu.make_async_copy(rows[q], accsh.at[didx[q]],
                                  ssem[q]).wait()

        if K > 0:
            def pbody(p, _):
                for q in range(NB4):
                    j = NB4 * p + q

                    @pl.when(j < K)
                    def _(j=j, q=q):
                        wait_g(q)
                        pltpu.async_copy(rows[q], accsh.at[didx[q]],
                                         ssem[q], add=True)

                        @pl.when(j + NB4 < K)
                        def _(j=j, q=q):
                            wait_s(q)
                            stage(j + NB4, q)
                return _
            lax.fori_loop(0, P, pbody, None)
            for q in range(min(NB4, K)):
                wait_s(q)

        if TL:
            t0 = e0 + K * C
            pltpu.sync_copy(s_h.at[pl.ds(t0, TL)], sT)
            pltpu.sync_copy(d_h.at[pl.ds(t0, TL)], dT)
            pltpu.async_copy(tab.at[sT], rows[0].at[pl.ds(0, TL)],
                             gsem[0]).wait()
            pltpu.sync_copy(rows[0].at[pl.ds(0, TL)], accsh.at[dT],
                            add=True)

        plsc.subcore_barrier()

        def ol(i, _):
            pltpu.async_copy(accsh.at[pl.ds(r0 + i * 32, 32)],
                             out.at[cid, pl.ds(r0 + i * 32, 32)], osem)
            return _
        lax.fori_loop(0, rpt // 32, ol, None)

        def od(i, _):
            pltpu.make_async_copy(
                accsh.at[pl.ds(r0, 32)],
                out.at[cid, pl.ds(r0, 32)], osem).wait()
            return _
        lax.fori_loop(0, rpt // 32, od, None)

    return k(table, src, dst)


def _sc_spmm_small2(tab1, tab2, s1, d1, s2, d2, ndst):
    E = s1.shape[0]
    eper = E // NW
    C = 128
    K = eper // C
    TL = eper - K * C
    assert s2.shape[0] == E and E % NW == 0 and TL % 8 == 0
    assert K == 2 and ndst % (NS * 32) == 0

    scratch = []
    for _ in range(2):
        scratch += [pltpu.VMEM((C,), jnp.int32),
                    pltpu.VMEM((C,), jnp.int32),
                    pltpu.VMEM((C, 128), F32)]
    scratch += [pltpu.VMEM((max(TL, 8),), jnp.int32),
                pltpu.VMEM((max(TL, 8),), jnp.int32),
                pltpu.VMEM((32, 128), F32),
                pltpu.VMEM_SHARED((ndst, 128), F32),
                pltpu.VMEM_SHARED((ndst, 128), F32)]
    scratch += [pltpu.SemaphoreType.DMA] * 5

    @functools.partial(
        pl.kernel,
        out_type=(jax.ShapeDtypeStruct((NC, ndst, 128), F32),
                  jax.ShapeDtypeStruct((NC, ndst, 128), F32)),
        mesh=_sc_mesh(),
        scratch_types=scratch,
    )
    def k(t1, t2, sa1, da1, sa2, da2, o1, o2, *sc):
        (s0b, d0b, r0b, s1b, d1b, r1b, sT, dT, zbuf,
         acc1, acc2, g0, g1, ss0, ss1, osem) = sc
        cid = lax.axis_index("c")
        sid = lax.axis_index("s")
        wid = sid * NC + cid
        e0 = wid * eper

        def zb(i, _):
            zbuf[i // 8, pl.ds((i % 8) * LANES, LANES)] = (
                jnp.zeros((LANES,), F32))
            return _
        lax.fori_loop(0, 256, zb, None)

        rpt = ndst // NS
        r0 = sid * rpt

        def phase(tab, s_h, d_h, accsh):
            pltpu.sync_copy(s_h.at[pl.ds(e0, C)], s0b)
            pltpu.sync_copy(d_h.at[pl.ds(e0, C)], d0b)
            pltpu.async_copy(tab.at[s0b], r0b, g0)
            pltpu.sync_copy(s_h.at[pl.ds(e0 + C, C)], s1b)
            pltpu.sync_copy(d_h.at[pl.ds(e0 + C, C)], d1b)
            pltpu.async_copy(tab.at[s1b], r1b, g1)

            def zl(i, _):
                pltpu.sync_copy(zbuf, accsh.at[pl.ds(r0 + i * 32, 32)])
                return _
            lax.fori_loop(0, rpt // 32, zl, None)
            plsc.subcore_barrier()

            pltpu.make_async_copy(tab.at[s0b], r0b, g0).wait()
            pltpu.async_copy(r0b, accsh.at[d0b], ss0, add=True)
            pltpu.make_async_copy(tab.at[s1b], r1b, g1).wait()
            pltpu.async_copy(r1b, accsh.at[d1b], ss1, add=True)

            t0 = e0 + K * C
            pltpu.sync_copy(s_h.at[pl.ds(t0, TL)], sT)
            pltpu.sync_copy(d_h.at[pl.ds(t0, TL)], dT)
            pltpu.make_async_copy(r0b, accsh.at[d0b], ss0).wait()
            pltpu.async_copy(tab.at[sT], r0b.at[pl.ds(0, TL)], g0).wait()
            pltpu.sync_copy(r0b.at[pl.ds(0, TL)], accsh.at[dT], add=True)
            pltpu.make_async_copy(r1b, accsh.at[d1b], ss1).wait()
            plsc.subcore_barrier()

        phase(t1, sa1, da1, acc1)
        phase(t2, sa2, da2, acc2)

        for accsh, out in ((acc1, o1), (acc2, o2)):
            def ol(i, _, accsh=accsh, out=out):
                pltpu.async_copy(accsh.at[pl.ds(r0 + i * 32, 32)],
                                 out.at[cid, pl.ds(r0 + i * 32, 32)], osem)
                return _
            lax.fori_loop(0, rpt // 32, ol, None)
        for accsh, out in ((acc1, o1), (acc2, o2)):
            def od(i, _, accsh=accsh, out=out):
                pltpu.make_async_copy(
                    accsh.at[pl.ds(r0, 32)],
                    out.at[cid, pl.ds(r0, 32)], osem).wait()
                return _
            lax.fori_loop(0, rpt // 32, od, None)

    return k(tab1, tab2, s1, d1, s2, d2)


def _gru_pool(accs, dinvs, xr, fw_x, G8s, g1s, g0, WhhT, bhh):
    n = accs[0].shape[1]
    na, nd = len(accs), len(dinvs)
    has_x = xr is not None
    BN = 1024
    assert n % BN == 0
    nb = n // BN
    nch = na + has_x

    def body(*refs):
        p = 0
        acc_r = refs[p:p + na]; p += na
        dv_r = refs[p:p + nd]; p += nd
        x_r = refs[p] if has_x else None
        p += 1 if has_x else 0
        G_r = refs[p]; p += 1
        g1_r = refs[p:p + na]; p += na
        g0_r, whh_r, bhh_r, out_ref = refs[p], refs[p + 1], refs[p + 2], \
            refs[p + 3]

        us = []
        for i in range(na):
            s = acc_r[i][0] + acc_r[i][1]
            if i < nd:
                sc = dv_r[i][0, :]
            else:
                sc = 1.0 / jnp.maximum(s[:, 112], 1.0)
            us.append(s * sc[:, None])

        base = jnp.broadcast_to(g0_r[...], (BN, 384))
        for i in range(na):
            base = base + us[i][:, 112:113] * g1_r[i][...]

        mats = list(us) + ([x_r[...]] if has_x else [])
        fws = [7] * na + ([fw_x] if has_x else [])
        BF = jnp.bfloat16
        Gfull = G_r[...].astype(BF)
        whh_bf = whh_r[...].astype(BF)
        h = jnp.zeros((BN, 128), F32)
        for t in range(16):
            if nch == 1:
                xin = mats[0][:, fws[0] * t:fws[0] * t + 8]
            else:
                xin = jnp.concatenate(
                    [m[:, fw * t:fw * t + 8] for m, fw in zip(mats, fws)],
                    axis=1)
            gi = base + jnp.dot(xin.astype(BF), Gfull,
                                preferred_element_type=F32)
            gh = jnp.dot(h.astype(BF), whh_bf, preferred_element_type=F32) \
                + bhh_r[...]
            r = 0.5 * jnp.tanh(0.5 * (gi[:, :128] + gh[:, :128])) + 0.5
            z = 0.5 * jnp.tanh(0.5 * (gi[:, 128:256] + gh[:, 128:256])) + 0.5
            nn2 = jnp.tanh(gi[:, 256:384] + r * gh[:, 256:384])
            h = nn2 + z * (h - nn2)

        ps = jnp.sum(h, axis=0, keepdims=True)

        @pl.when(pl.program_id(0) == 0)
        def _():
            out_ref[...] = ps

        @pl.when(pl.program_id(0) != 0)
        def _():
            out_ref[...] = out_ref[...] + ps

    in_specs = []
    args = []
    for a in accs:
        in_specs.append(pl.BlockSpec((NC, BN, 128), lambda b: (0, b, 0)))
        args.append(a)
    for dv in dinvs:
        in_specs.append(pl.BlockSpec((8, BN), lambda b: (0, b)))
        args.append(dv)
    if has_x:
        in_specs.append(pl.BlockSpec((BN, 128), lambda b: (b, 0)))
        args.append(xr)
    Gstack = jnp.concatenate(G8s, axis=0)
    in_specs.append(pl.BlockSpec((8 * nch, 384), lambda b: (0, 0)))
    args.append(Gstack)
    for g1 in g1s:
        in_specs.append(pl.BlockSpec((1, 384), lambda b: (0, 0)))
        args.append(g1)
    in_specs.append(pl.BlockSpec((1, 384), lambda b: (0, 0)))
    args.append(g0)
    in_specs.append(pl.BlockSpec((128, 384), lambda b: (0, 0)))
    args.append(WhhT)
    in_specs.append(pl.BlockSpec((1, 384), lambda b: (0, 0)))
    args.append(bhh)

    return pl.pallas_call(
        body,
        grid=(nb,),
        in_specs=in_specs,
        out_specs=pl.BlockSpec((1, 128), lambda b: (0, 0)),
        out_shape=jax.ShapeDtypeStruct((1, 128), F32),
    )(*args)


def _final(pa, pd, pb, Wh, bh, n_a, n_d, n_b):
    def body(pa_r, pd_r, pb_r, wh_r, bh_r, o_r):
        o_r[...] = (
            jnp.dot(pa_r[...] * (1.0 / n_a), wh_r[0:128],
                    preferred_element_type=F32)
            + jnp.dot(pd_r[...] * (1.0 / n_d), wh_r[128:256],
                      preferred_element_type=F32)
            + jnp.dot(pb_r[...] * (1.0 / n_b), wh_r[256:384],
                      preferred_element_type=F32)
            + bh_r[...])

    return pl.pallas_call(
        body,
        out_shape=jax.ShapeDtypeStruct((1, 256), F32),
    )(pa, pd, pb, Wh, bh)


def kernel(x_attk, x_def, x_ball, Wpa, bpa, Wpd, bpd, Wpb, bpb,
           W_gcn_aa, b_gcn_aa, W_gcn_dd, b_gcn_dd,
           Wl_ad, bl_ad, Wr_ad, Wl_ab, bl_ab, Wr_ab, Wl_db, bl_db, Wr_db,
           Wih, Whh, bih, bhh, Wh, bh,
           ei_aa_src, ei_aa_dst, ei_dd_src, ei_dd_dst, ei_ad_src, ei_ad_dst,
           ei_ab_src, ei_ab_dst, ei_db_src, ei_db_dst):
    T, n_a = x_attk.shape[0], x_attk.shape[1]
    n_d, n_b = x_def.shape[1], x_ball.shape[1]

    xa = jnp.transpose(x_attk, (1, 0, 2)).reshape(n_a, 7 * T)
    xd = jnp.transpose(x_def, (1, 0, 2)).reshape(n_d, 7 * T)
    xb = jnp.transpose(x_ball, (1, 0, 2)).reshape(n_b, 4 * T)
    xa_p = jnp.pad(xa, ((0, 0), (0, 128 - 7 * T)))
    xd_p = jnp.pad(xd, ((0, 0), (0, 128 - 7 * T)))
    xb_p = jnp.pad(xb, ((0, 0), (0, 128 - 4 * T)))

    hists = _sc_counts(ei_aa_dst, ei_dd_dst, n_a, n_d)

    ysa, yra, dv_a = _prep(hists[:, :n_a], xa_p)
    ysd, yrd, dv_d = _prep(hists[:, n_a:], xd_p)

    acc_aa = _sc_spmm_one(ysa, ei_aa_src, ei_aa_dst, n_a)
    acc_dd = _sc_spmm_one(ysd, ei_dd_src, ei_dd_dst, n_d)
    acc_ad = _sc_spmm_one(yra, ei_ad_src, ei_ad_dst, n_d)
    acc_ab, acc_db = _sc_spmm_small2(yra, yrd, ei_ab_src, ei_ab_dst,
                                     ei_db_src, ei_db_dst, n_b)

    WihT = Wih.T
    WhhT = Whh.T

    def pad8(G):
        return jnp.pad(G, ((0, 8 - G.shape[0]), (0, 0)))

    def row(v):
        return v.reshape(1, -1)

    Ga = pad8((Wpa @ W_gcn_aa) @ WihT)
    g1a = row((bpa @ W_gcn_aa) @ WihT)
    g0a = row(b_gcn_aa @ WihT + bih)
    Gdd = pad8((Wpd @ W_gcn_dd) @ WihT)
    g1dd = row((bpd @ W_gcn_dd) @ WihT)
    Gad = pad8((Wpa @ Wl_ad) @ WihT)
    g1ad = row((bpa @ Wl_ad) @ WihT)
    Gdr = pad8((Wpd @ Wr_ad) @ WihT)
    g0d = row((b_gcn_dd + bl_ad + bpd @ Wr_ad) @ WihT + bih)
    Gab = pad8((Wpa @ Wl_ab) @ WihT)
    g1ab = row((bpa @ Wl_ab) @ WihT)
    Gdb = pad8((Wpd @ Wl_db) @ WihT)
    g1db = row((bpd @ Wl_db) @ WihT)
    Wr_b = Wr_ab + Wr_db
    Gbr = pad8((Wpb @ Wr_b) @ WihT)
    g0b = row((bl_ab + bl_db + bpb @ Wr_b) @ WihT + bih)
    bhh_r = row(bhh)

    pa = _gru_pool([acc_aa], [dv_a], None, 0, [Ga], [g1a], g0a, WhhT, bhh_r)
    pd = _gru_pool([acc_dd, acc_ad], [dv_d], xd_p, 7, [Gdd, Gad, Gdr],
                   [g1dd, g1ad], g0d, WhhT, bhh_r)
    pb = _gru_pool([acc_ab, acc_db], [], xb_p, 4, [Gab, Gdb, Gbr],
                   [g1ab, g1db], g0b, WhhT, bhh_r)

    H = _final(pa, pd, pb, Wh, row(bh), n_a, n_d, n_b)
    return H.reshape(256)


# speedup vs baseline: 1.1284x; 1.1284x over previous
import functools

import jax
import jax.numpy as jnp
from jax import lax
from jax.experimental import pallas as pl
from jax.experimental.pallas import tpu as pltpu
from jax.experimental.pallas import tpu_sc as plsc

F32 = jnp.float32
NC, NS = 2, 16
NW = NC * NS
LANES = 16


def _sc_mesh():
    return plsc.VectorSubcoreMesh(
        core_axis_name="c", subcore_axis_name="s",
        num_cores=NC, num_subcores=NS)


def _sc_counts(d1, d2, n1, n2):
    NH = n1 + n2
    E1, E2 = d1.shape[0] // NW, d2.shape[0] // NW
    assert E1 % LANES == 0 and E2 % LANES == 0 and NH % LANES == 0

    @functools.partial(
        pl.kernel,
        out_type=jax.ShapeDtypeStruct((NW, NH), F32),
        mesh=_sc_mesh(),
        scratch_types=[pltpu.VMEM((E1,), jnp.int32),
                       pltpu.VMEM((E2,), jnp.int32),
                       pltpu.VMEM((NH,), F32),
                       pltpu.SemaphoreType.DMA],
        compiler_params=pltpu.CompilerParams(needs_layout_passes=False),
    )
    def k(d1_hbm, d2_hbm, out_hbm, idx1, idx2, hist, sem):
        wid = lax.axis_index("s") * NC + lax.axis_index("c")
        pltpu.async_copy(d1_hbm.at[pl.ds(wid * E1, E1)], idx1, sem)
        pltpu.async_copy(d2_hbm.at[pl.ds(wid * E2, E2)], idx2, sem)

        def zero(i, _):
            hist[pl.ds(i * LANES, LANES)] = jnp.zeros((LANES,), F32)
            return _
        lax.fori_loop(0, NH // LANES, zero, None)

        pltpu.make_async_copy(d1_hbm.at[pl.ds(wid * E1, E1)], idx1,
                              sem).wait()
        pltpu.make_async_copy(d2_hbm.at[pl.ds(wid * E2, E2)], idx2,
                              sem).wait()
        ones = jnp.ones((LANES,), F32)
        for (idxb, eper, off) in ((idx1, E1, 0), (idx2, E2, n1)):
            def inner(j, _, idxb=idxb, off=off):
                iv = idxb[pl.ds(j * LANES, LANES)]
                if off:
                    iv = iv + off
                plsc.addupdate_scatter(hist, [iv], ones)
                return _
            lax.fori_loop(0, eper // LANES, inner, None)

        pltpu.sync_copy(hist, out_hbm.at[wid])

    return k(d1, d2)


def _prep(hists, x_pad):
    n = x_pad.shape[0]
    BN = 256
    assert n % BN == 0

    def body(h_ref, x_ref, ys_ref, yr_ref, dv_ref):
        cnt = jnp.sum(h_ref[...], axis=0)
        dv = jnp.where(cnt > 0, lax.rsqrt(cnt), 0.0)
        x = x_ref[...]
        is112 = lax.broadcasted_iota(jnp.int32, x.shape, 1) == 112
        yr_ref[...] = jnp.where(is112, 1.0, x)
        ys_ref[...] = jnp.where(is112, dv[:, None], x * dv[:, None])
        dv_ref[...] = jnp.broadcast_to(dv[None, :], (8, BN))

    return pl.pallas_call(
        body,
        grid=(n // BN,),
        in_specs=[pl.BlockSpec((NW, BN), lambda b: (0, b)),
                  pl.BlockSpec((BN, 128), lambda b: (b, 0))],
        out_specs=[pl.BlockSpec((BN, 128), lambda b: (b, 0)),
                   pl.BlockSpec((BN, 128), lambda b: (b, 0)),
                   pl.BlockSpec((8, BN), lambda b: (0, b))],
        out_shape=[jax.ShapeDtypeStruct((n, 128), F32),
                   jax.ShapeDtypeStruct((n, 128), F32),
                   jax.ShapeDtypeStruct((8, n), F32)],
    )(hists, x_pad)


def _sc_spmm_one(table, src, dst, ndst):
    E = src.shape[0]
    eper = E // NW
    C = 128
    K = eper // C
    TL = eper - K * C
    NB4 = 2
    P = (K + NB4 - 1) // NB4
    assert E % NW == 0 and TL % 8 == 0 and ndst % (NS * 32) == 0

    scratch = []
    for _ in range(NB4):
        scratch += [pltpu.VMEM((C,), jnp.int32),
                    pltpu.VMEM((C,), jnp.int32),
                    pltpu.VMEM((C, 128), F32)]
    scratch += [pltpu.VMEM((max(TL, 8),), jnp.int32),
                pltpu.VMEM((max(TL, 8),), jnp.int32),
                pltpu.VMEM((32, 128), F32),
                pltpu.VMEM_SHARED((ndst, 128), F32)]
    scratch += [pltpu.SemaphoreType.DMA] * (2 * NB4 + 1)

    @functools.partial(
        pl.kernel,
        out_type=jax.ShapeDtypeStruct((NC, ndst, 128), F32),
        mesh=_sc_mesh(),
        scratch_types=scratch,
    )
    def k(tab, s_h, d_h, out, *sc):
        sidx = [sc[3 * q + 0] for q in range(NB4)]
        didx = [sc[3 * q + 1] for q in range(NB4)]
        rows = [sc[3 * q + 2] for q in range(NB4)]
        sT, dT, zbuf, accsh = sc[3 * NB4:3 * NB4 + 4]
        gsem = list(sc[3 * NB4 + 4:3 * NB4 + 4 + NB4])
        ssem = list(sc[3 * NB4 + 4 + NB4:3 * NB4 + 4 + 2 * NB4])
        osem = sc[3 * NB4 + 4 + 2 * NB4]

        cid = lax.axis_index("c")
        sid = lax.axis_index("s")
        wid = sid * NC + cid
        e0 = wid * eper

        def stage(j, q):
            pltpu.sync_copy(s_h.at[pl.ds(e0 + j * C, C)], sidx[q])
            pltpu.sync_copy(d_h.at[pl.ds(e0 + j * C, C)], didx[q])
            pltpu.async_copy(tab.at[sidx[q]], rows[q], gsem[q])

        for q in range(min(NB4, K)):
            stage(q, q)

        def zb(i, _):
            zbuf[i // 8, pl.ds((i % 8) * LANES, LANES)] = (
                jnp.zeros((LANES,), F32))
            return _
        lax.fori_loop(0, 256, zb, None)

        rpt = ndst // NS
        r0 = sid * rpt

        def zl(i, _):
            pltpu.sync_copy(zbuf, accsh.at[pl.ds(r0 + i * 32, 32)])
            return _
        lax.fori_loop(0, rpt // 32, zl, None)
        plsc.subcore_barrier()

        def wait_g(q):
            pltpu.make_async_copy(tab.at[sidx[q]], rows[q], gsem[q]).wait()

        def wait_s(q):
            pltpu.make_async_copy(rows[q], accsh.at[didx[q]],
                                  ssem[q]).wait()

        if K > 0:
            def pbody(p, _):
                for q in range(NB4):
                    j = NB4 * p + q

                    @pl.when(j < K)
                    def _(j=j, q=q):
                        wait_g(q)
                        pltpu.async_copy(rows[q], accsh.at[didx[q]],
                                         ssem[q], add=True)

                        @pl.when(j + NB4 < K)
                        def _(j=j, q=q):
                            wait_s(q)
                            stage(j + NB4, q)
                return _
            lax.fori_loop(0, P, pbody, None)
            for q in range(min(NB4, K)):
                wait_s(q)

        if TL:
            t0 = e0 + K * C
            pltpu.sync_copy(s_h.at[pl.ds(t0, TL)], sT)
            pltpu.sync_copy(d_h.at[pl.ds(t0, TL)], dT)
            pltpu.async_copy(tab.at[sT], rows[0].at[pl.ds(0, TL)],
                             gsem[0]).wait()
            pltpu.sync_copy(rows[0].at[pl.ds(0, TL)], accsh.at[dT],
                            add=True)

        plsc.subcore_barrier()

        def ol(i, _):
            pltpu.async_copy(accsh.at[pl.ds(r0 + i * 32, 32)],
                             out.at[cid, pl.ds(r0 + i * 32, 32)], osem)
            return _
        lax.fori_loop(0, rpt // 32, ol, None)

        def od(i, _):
            pltpu.make_async_copy(
                accsh.at[pl.ds(r0, 32)],
                out.at[cid, pl.ds(r0, 32)], osem).wait()
            return _
        lax.fori_loop(0, rpt // 32, od, None)

    return k(table, src, dst)


def _gru_pool(accs, dinvs, xr, fw_x, G8s, g1s, g0, WhhT, bhh):
    n = accs[0].shape[1]
    na, nd = len(accs), len(dinvs)
    has_x = xr is not None
    BN = 1024
    assert n % BN == 0
    nb = n // BN
    nch = na + has_x

    def body(*refs):
        p = 0
        acc_r = refs[p:p + na]; p += na
        dv_r = refs[p:p + nd]; p += nd
        x_r = refs[p] if has_x else None
        p += 1 if has_x else 0
        G_r = refs[p]; p += 1
        g1_r = refs[p:p + na]; p += na
        g0_r, whh_r, bhh_r, out_ref = refs[p], refs[p + 1], refs[p + 2], \
            refs[p + 3]

        us = []
        for i in range(na):
            s = acc_r[i][0] + acc_r[i][1]
            if i < nd:
                sc = dv_r[i][0, :]
            else:
                sc = 1.0 / jnp.maximum(s[:, 112], 1.0)
            us.append(s * sc[:, None])

        base = jnp.broadcast_to(g0_r[...], (BN, 384))
        for i in range(na):
            base = base + us[i][:, 112:113] * g1_r[i][...]

        mats = list(us) + ([x_r[...]] if has_x else [])
        fws = [7] * na + ([fw_x] if has_x else [])
        BF = jnp.bfloat16
        Gfull = G_r[...].astype(BF)
        whh_bf = whh_r[...].astype(BF)
        h = jnp.zeros((BN, 128), F32)
        for t in range(16):
            if nch == 1:
                xin = mats[0][:, fws[0] * t:fws[0] * t + 8]
            else:
                xin = jnp.concatenate(
                    [m[:, fw * t:fw * t + 8] for m, fw in zip(mats, fws)],
                    axis=1)
            gi = base + jnp.dot(xin.astype(BF), Gfull,
                                preferred_element_type=F32)
            gh = jnp.dot(h.astype(BF), whh_bf, preferred_element_type=F32) \
                + bhh_r[...]
            r = 0.5 * jnp.tanh(0.5 * (gi[:, :128] + gh[:, :128])) + 0.5
            z = 0.5 * jnp.tanh(0.5 * (gi[:, 128:256] + gh[:, 128:256])) + 0.5
            nn2 = jnp.tanh(gi[:, 256:384] + r * gh[:, 256:384])
            h = nn2 + z * (h - nn2)

        ps = jnp.sum(h, axis=0, keepdims=True)

        @pl.when(pl.program_id(0) == 0)
        def _():
            out_ref[...] = ps

        @pl.when(pl.program_id(0) != 0)
        def _():
            out_ref[...] = out_ref[...] + ps

    in_specs = []
    args = []
    for a in accs:
        in_specs.append(pl.BlockSpec((NC, BN, 128), lambda b: (0, b, 0)))
        args.append(a)
    for dv in dinvs:
        in_specs.append(pl.BlockSpec((8, BN), lambda b: (0, b)))
        args.append(dv)
    if has_x:
        in_specs.append(pl.BlockSpec((BN, 128), lambda b: (b, 0)))
        args.append(xr)
    Gstack = jnp.concatenate(G8s, axis=0)
    in_specs.append(pl.BlockSpec((8 * nch, 384), lambda b: (0, 0)))
    args.append(Gstack)
    for g1 in g1s:
        in_specs.append(pl.BlockSpec((1, 384), lambda b: (0, 0)))
        args.append(g1)
    in_specs.append(pl.BlockSpec((1, 384), lambda b: (0, 0)))
    args.append(g0)
    in_specs.append(pl.BlockSpec((128, 384), lambda b: (0, 0)))
    args.append(WhhT)
    in_specs.append(pl.BlockSpec((1, 384), lambda b: (0, 0)))
    args.append(bhh)

    return pl.pallas_call(
        body,
        grid=(nb,),
        in_specs=in_specs,
        out_specs=pl.BlockSpec((1, 128), lambda b: (0, 0)),
        out_shape=jax.ShapeDtypeStruct((1, 128), F32),
    )(*args)


def _final(pa, pd, pb, Wh, bh, n_a, n_d, n_b):
    def body(pa_r, pd_r, pb_r, wh_r, bh_r, o_r):
        o_r[...] = (
            jnp.dot(pa_r[...] * (1.0 / n_a), wh_r[0:128],
                    preferred_element_type=F32)
            + jnp.dot(pd_r[...] * (1.0 / n_d), wh_r[128:256],
                      preferred_element_type=F32)
            + jnp.dot(pb_r[...] * (1.0 / n_b), wh_r[256:384],
                      preferred_element_type=F32)
            + bh_r[...])

    return pl.pallas_call(
        body,
        out_shape=jax.ShapeDtypeStruct((1, 256), F32),
    )(pa, pd, pb, Wh, bh)


def kernel(x_attk, x_def, x_ball, Wpa, bpa, Wpd, bpd, Wpb, bpb,
           W_gcn_aa, b_gcn_aa, W_gcn_dd, b_gcn_dd,
           Wl_ad, bl_ad, Wr_ad, Wl_ab, bl_ab, Wr_ab, Wl_db, bl_db, Wr_db,
           Wih, Whh, bih, bhh, Wh, bh,
           ei_aa_src, ei_aa_dst, ei_dd_src, ei_dd_dst, ei_ad_src, ei_ad_dst,
           ei_ab_src, ei_ab_dst, ei_db_src, ei_db_dst):
    T, n_a = x_attk.shape[0], x_attk.shape[1]
    n_d, n_b = x_def.shape[1], x_ball.shape[1]

    xa = jnp.transpose(x_attk, (1, 0, 2)).reshape(n_a, 7 * T)
    xd = jnp.transpose(x_def, (1, 0, 2)).reshape(n_d, 7 * T)
    xb = jnp.transpose(x_ball, (1, 0, 2)).reshape(n_b, 4 * T)
    xa_p = jnp.pad(xa, ((0, 0), (0, 128 - 7 * T)))
    xd_p = jnp.pad(xd, ((0, 0), (0, 128 - 7 * T)))
    xb_p = jnp.pad(xb, ((0, 0), (0, 128 - 4 * T)))

    hists = _sc_counts(ei_aa_dst, ei_dd_dst, n_a, n_d)

    ysa, yra, dv_a = _prep(hists[:, :n_a], xa_p)
    ysd, yrd, dv_d = _prep(hists[:, n_a:], xd_p)

    acc_aa = _sc_spmm_one(ysa, ei_aa_src, ei_aa_dst, n_a)
    acc_dd = _sc_spmm_one(ysd, ei_dd_src, ei_dd_dst, n_d)
    acc_ad = _sc_spmm_one(yra, ei_ad_src, ei_ad_dst, n_d)
    acc_ab = _sc_spmm_one(yra, ei_ab_src, ei_ab_dst, n_b)
    acc_db = _sc_spmm_one(yrd, ei_db_src, ei_db_dst, n_b)

    WihT = Wih.T
    WhhT = Whh.T

    def pad8(G):
        return jnp.pad(G, ((0, 8 - G.shape[0]), (0, 0)))

    def row(v):
        return v.reshape(1, -1)

    Ga = pad8((Wpa @ W_gcn_aa) @ WihT)
    g1a = row((bpa @ W_gcn_aa) @ WihT)
    g0a = row(b_gcn_aa @ WihT + bih)
    Gdd = pad8((Wpd @ W_gcn_dd) @ WihT)
    g1dd = row((bpd @ W_gcn_dd) @ WihT)
    Gad = pad8((Wpa @ Wl_ad) @ WihT)
    g1ad = row((bpa @ Wl_ad) @ WihT)
    Gdr = pad8((Wpd @ Wr_ad) @ WihT)
    g0d = row((b_gcn_dd + bl_ad + bpd @ Wr_ad) @ WihT + bih)
    Gab = pad8((Wpa @ Wl_ab) @ WihT)
    g1ab = row((bpa @ Wl_ab) @ WihT)
    Gdb = pad8((Wpd @ Wl_db) @ WihT)
    g1db = row((bpd @ Wl_db) @ WihT)
    Wr_b = Wr_ab + Wr_db
    Gbr = pad8((Wpb @ Wr_b) @ WihT)
    g0b = row((bl_ab + bl_db + bpb @ Wr_b) @ WihT + bih)
    bhh_r = row(bhh)

    pa = _gru_pool([acc_aa], [dv_a], None, 0, [Ga], [g1a], g0a, WhhT, bhh_r)
    pd = _gru_pool([acc_dd, acc_ad], [dv_d], xd_p, 7, [Gdd, Gad, Gdr],
                   [g1dd, g1ad], g0d, WhhT, bhh_r)
    pb = _gru_pool([acc_ab, acc_db], [], xb_p, 4, [Gab, Gdb, Gbr],
                   [g1ab, g1db], g0b, WhhT, bhh_r)

    H = _final(pa, pd, pb, Wh, row(bh), n_a, n_d, n_b)
    return H.reshape(256)
